# baseline (speedup 1.0000x reference)
import functools

import jax
import jax.numpy as jnp
from jax import lax
from jax.experimental import pallas as pl
from jax.experimental.pallas import tpu as pltpu
from jax.experimental.pallas import tpu_sc as plsc

N_GRAPH = 64
N_NODE = 128
N_FEAT = 9
N_ROW = 11
N_PAD = 12
HIDDEN = 768
LANES = 16
NC = 2
NS = 16
NW = NC * NS
GPW = N_GRAPH // NW
C = 4
NCHUNK = N_NODE // C
NT = GPW * NCHUNK
ROWS_PC = C * N_PAD
IDX_PW = GPW * N_NODE * N_PAD


def _build_kernel():
    mesh = plsc.VectorSubcoreMesh(core_axis_name="c", subcore_axis_name="s")

    @functools.partial(
        pl.kernel,
        mesh=mesh,
        compiler_params=pltpu.CompilerParams(use_tc_tiling_on_sc=False),
        out_type=jax.ShapeDtypeStruct((N_GRAPH, N_NODE + 1, HIDDEN), jnp.float32),
        scratch_types=[
            pltpu.VMEM((IDX_PW,), jnp.int32),
            pltpu.VMEM((2, ROWS_PC, HIDDEN), jnp.float32),
            pltpu.VMEM((2, C, HIDDEN), jnp.float32),
            pltpu.VMEM((1, HIDDEN), jnp.float32),
            pltpu.SemaphoreType.DMA,
            pltpu.SemaphoreType.DMA,
            pltpu.SemaphoreType.DMA,
            pltpu.SemaphoreType.DMA,
            pltpu.SemaphoreType.DMA,
        ],
    )
    def k(idx_hbm, table_hbm, tok_hbm, out_hbm,
          idxv, gbuf, rbuf, tokv, sem_p, sem_g0, sem_g1, sem_o0, sem_o1):
        wid = lax.axis_index("s") * NC + lax.axis_index("c")
        sem_g = (sem_g0, sem_g1)
        sem_o = (sem_o0, sem_o1)

        pltpu.async_copy(idx_hbm.at[pl.ds(wid * IDX_PW, IDX_PW)], idxv, sem_p)
        pltpu.async_copy(tok_hbm, tokv, sem_p)
        pltpu.make_async_copy(idx_hbm.at[pl.ds(0, IDX_PW)], idxv, sem_p).wait()
        pltpu.make_async_copy(tok_hbm, tokv, sem_p).wait()

        for gl in range(GPW):
            g = wid * GPW + gl
            pltpu.async_copy(tokv, out_hbm.at[g, pl.ds(0, 1)], sem_p)

        def fire_gather(t, b):
            pltpu.async_copy(
                table_hbm.at[idxv.at[pl.ds(t * ROWS_PC, ROWS_PC)]],
                gbuf.at[b], sem_g[b])

        def do_chunk(t, b):
            pltpu.make_async_copy(
                table_hbm.at[idxv.at[pl.ds(0, ROWS_PC)]],
                gbuf.at[b], sem_g[b]).wait()

            @pl.when(t + 1 < NT)
            def _():
                fire_gather(t + 1, 1 - b)

            @pl.when(t >= 2)
            def _():
                pltpu.make_async_copy(
                    rbuf.at[b], out_hbm.at[0, pl.ds(0, C)], sem_o[b]).wait()

            @pl.loop(0, C)
            def _node(i):
                @pl.loop(0, HIDDEN // LANES)
                def _col(j):
                    col = j * LANES
                    acc = gbuf[b, i * N_PAD, pl.ds(col, LANES)]
                    for f in range(1, N_ROW):
                        acc = acc + gbuf[b, i * N_PAD + f, pl.ds(col, LANES)]
                    rbuf[b, i, pl.ds(col, LANES)] = acc

            g = wid * GPW + lax.div(t, NCHUNK)
            node0 = lax.rem(t, NCHUNK) * C
            pltpu.async_copy(rbuf.at[b],
                             out_hbm.at[g, pl.ds(1 + node0, C)], sem_o[b])

        fire_gather(0, 0)

        @pl.loop(0, NT, step=2)
        def _pair(t0):
            do_chunk(t0, 0)
            do_chunk(t0 + 1, 1)

        pltpu.make_async_copy(rbuf.at[0], out_hbm.at[0, pl.ds(0, C)], sem_o[0]).wait()
        pltpu.make_async_copy(rbuf.at[1], out_hbm.at[0, pl.ds(0, C)], sem_o[1]).wait()
        for _ in range(GPW):
            pltpu.make_async_copy(tokv, out_hbm.at[0, pl.ds(0, 1)], sem_p).wait()

    return k


_KERNEL = _build_kernel()


def kernel(x, in_degree, out_degree, atom_table, in_deg_table, out_deg_table,
           graph_token):
    n_atom = atom_table.shape[0]
    n_in = in_deg_table.shape[0]
    x = x.astype(jnp.int32)
    ind = in_degree.astype(jnp.int32) + n_atom
    outd = out_degree.astype(jnp.int32) + n_atom + n_in
    pad = jnp.zeros(x.shape[:2] + (1,), jnp.int32)
    idx = jnp.concatenate([x, ind[..., None], outd[..., None], pad], axis=-1)
    table = jnp.concatenate([atom_table, in_deg_table, out_deg_table], axis=0)
    return _KERNEL(idx.reshape(-1), table, graph_token)


# speedup vs baseline: 1.2549x; 1.2549x over previous
import functools

import jax
import jax.numpy as jnp
from jax import lax
from jax.experimental import pallas as pl
from jax.experimental.pallas import tpu as pltpu
from jax.experimental.pallas import tpu_sc as plsc

N_GRAPH = 64
N_NODE = 128
N_FEAT = 9
HIDDEN = 768
LANES = 16
NC = 2
NS = 16
NW = NC * NS
GPW = N_GRAPH // NW
C = 8
NCHUNK = N_NODE // C


def _build_kernel():
    mesh = plsc.VectorSubcoreMesh(core_axis_name="c", subcore_axis_name="s")

    @functools.partial(
        pl.kernel,
        mesh=mesh,
        compiler_params=pltpu.CompilerParams(use_tc_tiling_on_sc=False),
        out_type=jax.ShapeDtypeStruct((N_GRAPH, N_NODE + 1, HIDDEN), jnp.float32),
        scratch_types=[
            pltpu.VMEM((C * N_FEAT,), jnp.int32),
            pltpu.VMEM((C,), jnp.int32),
            pltpu.VMEM((C,), jnp.int32),
            pltpu.VMEM((C * N_FEAT, HIDDEN), jnp.float32),
            pltpu.VMEM((C, HIDDEN), jnp.float32),
            pltpu.VMEM((C, HIDDEN), jnp.float32),
            pltpu.VMEM((C, HIDDEN), jnp.float32),
            pltpu.VMEM((1, HIDDEN), jnp.float32),
        ],
    )
    def k(x_hbm, ind_hbm, outd_hbm, atom_hbm, int_hbm, outt_hbm, tok_hbm,
          out_hbm, xidx, iidx, oidx, arows, irows, orows, res, tokv):
        wid = lax.axis_index("s") * NC + lax.axis_index("c")
        pltpu.sync_copy(tok_hbm, tokv)

        for gl in range(GPW):
            g = wid * GPW + gl
            pltpu.sync_copy(tokv, out_hbm.at[g, pl.ds(0, 1)])

            @pl.loop(0, NCHUNK)
            def _chunk(cix, g=g):
                node0 = cix * C
                pltpu.sync_copy(
                    x_hbm.at[pl.ds(g * (N_NODE * N_FEAT) + node0 * N_FEAT,
                                   C * N_FEAT)],
                    xidx)
                pltpu.sync_copy(ind_hbm.at[pl.ds(g * N_NODE + node0, C)], iidx)
                pltpu.sync_copy(outd_hbm.at[pl.ds(g * N_NODE + node0, C)], oidx)
                pltpu.sync_copy(atom_hbm.at[xidx], arows)
                pltpu.sync_copy(int_hbm.at[iidx], irows)
                pltpu.sync_copy(outt_hbm.at[oidx], orows)

                @pl.loop(0, C)
                def _node(i):
                    @pl.loop(0, HIDDEN // LANES)
                    def _col(j):
                        col = j * LANES
                        acc = arows[i * N_FEAT, pl.ds(col, LANES)]
                        for f in range(1, N_FEAT):
                            acc = acc + arows[i * N_FEAT + f, pl.ds(col, LANES)]
                        acc = acc + irows[i, pl.ds(col, LANES)]
                        acc = acc + orows[i, pl.ds(col, LANES)]
                        res[i, pl.ds(col, LANES)] = acc

                pltpu.sync_copy(res, out_hbm.at[g, pl.ds(1 + node0, C)])

    return k


_KERNEL = _build_kernel()


def kernel(x, in_degree, out_degree, atom_table, in_deg_table, out_deg_table,
           graph_token):
    x = x.reshape(-1).astype(jnp.int32)
    ind = in_degree.reshape(-1).astype(jnp.int32)
    outd = out_degree.reshape(-1).astype(jnp.int32)
    return _KERNEL(x, ind, outd, atom_table, in_deg_table, out_deg_table,
                   graph_token)
